# Initial kernel scaffold; baseline (speedup 1.0000x reference)
import jax
import jax.numpy as jnp
from jax.experimental import pallas as pl


def kernel(x, trans_row, trans_col, trans_val, spiral_idx, W, b):
    raise NotImplementedError("write your pallas kernel here")


# speedup vs baseline: 1.9235x; 1.9235x over previous
import functools

import jax
import jax.numpy as jnp
from jax import lax
from jax.experimental import pallas as pl
from jax.experimental.pallas import tpu as pltpu
from jax.experimental.pallas import tpu_sc as plsc

_B = 16
_N_IN = 10000
_N_OUT = 20000
_C = 128
_SEQ = 9
_NNZ = 60000

_NC = 2
_NS = 16
_NW = _NC * _NS
_RPW = 640
_N_OUT_PAD = _RPW * _NW
_CH = 128
_NNZ_PAD = ((_NNZ + 2 * _CH - 1) // _CH) * _CH
_L = 16

_TOT = _N_OUT * _SEQ
_GCH = 128
_NCHUNK = (_TOT + _GCH - 1) // _GCH
_CPW = (_NCHUNK + _NW - 1) // _NW
_LAST = _TOT - _GCH


def _worker_id():
    return lax.axis_index("s") * _NC + lax.axis_index("c")


def _pool_body(xf, cols, rows, vals, bnd, zsrc, pooled,
               colb, rowb, valb, idxb, gb, acc, bb, sem):
    w = _worker_id()
    base_row = w * _RPW
    pltpu.sync_copy(bnd.at[pl.ds(pl.multiple_of(w * _L, 8), _L)], bb)
    bv = bb[...]
    lo = bv[0]
    hi = bv[1]
    a_lo = pl.multiple_of((lo // 8) * 8, 8)
    nch = (hi - a_lo + _CH - 1) // _CH
    lanes = lax.iota(jnp.int32, _L)

    def batch_body(b, carry):
        pltpu.sync_copy(zsrc, acc)
        boff = b * _N_IN

        def chunk_body(i, c2):
            p0 = pl.multiple_of(a_lo + i * _CH, 8)
            pltpu.sync_copy(cols.at[pl.ds(p0, _CH)], colb)
            pltpu.sync_copy(rows.at[pl.ds(p0, _CH)], rowb)
            pltpu.sync_copy(vals.at[pl.ds(p0, _CH)], valb)

            def idx_body(g, c3):
                idxb[pl.ds(g * _L, _L)] = colb[pl.ds(g * _L, _L)] + boff
                return c3
            lax.fori_loop(0, _CH // _L, idx_body, None)
            pltpu.async_copy(xf.at[idxb], gb, sem).wait()

            def group_body(g, c3):
                q0 = g * _L
                val_v = valb[pl.ds(q0, _L)]
                row_v = rowb[pl.ds(q0, _L)]
                pos_v = (p0 + q0) + lanes
                m = (pos_v >= lo) & (pos_v < hi)
                val_m = jnp.where(m, val_v, 0.0)
                rloc_v = jnp.clip(row_v - base_row, 0, _RPW - 1)
                for l in range(_L):
                    vs = val_m[l]
                    rs = rloc_v[l]
                    for cg in range(_C // _L):
                        gv = gb[q0 + l, pl.ds(cg * _L, _L)]
                        plsc.addupdate(acc.at[rs, pl.ds(cg * _L, _L)], gv * vs)
                return c3
            lax.fori_loop(0, _CH // _L, group_body, None)
            return c2
        lax.fori_loop(0, nch, chunk_body, None)
        pltpu.sync_copy(
            acc,
            pooled.at[pl.ds(pl.multiple_of(b * _N_OUT_PAD + base_row, 8), _RPW)])
        return carry
    lax.fori_loop(0, _B, batch_body, None)


def _gather_body(pooled, sidx, feat, ib, idxb, gb, sem):
    w = _worker_id()

    def chunk_body(i, carry):
        k = i * _NW + w

        @pl.when(k < _NCHUNK)
        def _():
            s_k = pl.multiple_of(jnp.minimum(k * _GCH, _LAST), 8)
            pltpu.sync_copy(sidx.at[pl.ds(s_k, _GCH)], ib)

            def batch_body(b, c2):
                def idx_body(g, c3):
                    idxb[pl.ds(g * _L, _L)] = (ib[pl.ds(g * _L, _L)]
                                               + b * _N_OUT_PAD)
                    return c3
                lax.fori_loop(0, _GCH // _L, idx_body, None)
                pltpu.async_copy(pooled.at[idxb], gb, sem).wait()
                pltpu.sync_copy(
                    gb,
                    feat.at[pl.ds(pl.multiple_of(b * _TOT + s_k, 8), _GCH)])
                return c2
            lax.fori_loop(0, _B, batch_body, None)
        return carry
    lax.fori_loop(0, _CPW, chunk_body, None)


def _sc_pool(xf, cols, rows, vals, bnd, zsrc):
    mesh = plsc.VectorSubcoreMesh(core_axis_name="c", subcore_axis_name="s")
    f = pl.kernel(
        _pool_body,
        out_type=jax.ShapeDtypeStruct((_B * _N_OUT_PAD, _C), jnp.float32),
        mesh=mesh,
        scratch_types=[
            pltpu.VMEM((_CH,), jnp.int32),
            pltpu.VMEM((_CH,), jnp.int32),
            pltpu.VMEM((_CH,), jnp.float32),
            pltpu.VMEM((_CH,), jnp.int32),
            pltpu.VMEM((_CH, _C), jnp.float32),
            pltpu.VMEM((_RPW, _C), jnp.float32),
            pltpu.VMEM((16,), jnp.int32),
            pltpu.SemaphoreType.DMA,
        ],
    )
    return f(xf, cols, rows, vals, bnd, zsrc)


def _sc_gather(pooled, sidx):
    mesh = plsc.VectorSubcoreMesh(core_axis_name="c", subcore_axis_name="s")
    f = pl.kernel(
        _gather_body,
        out_type=jax.ShapeDtypeStruct((_B * _TOT, _C), jnp.float32),
        mesh=mesh,
        scratch_types=[
            pltpu.VMEM((_GCH,), jnp.int32),
            pltpu.VMEM((_GCH,), jnp.int32),
            pltpu.VMEM((_GCH, _C), jnp.float32),
            pltpu.SemaphoreType.DMA,
        ],
    )
    return f(pooled, sidx)


def _mm_body(f_ref, w_ref, b_ref, o_ref):
    acc = jnp.dot(f_ref[...], w_ref[...], preferred_element_type=jnp.float32)
    o_ref[...] = jnp.maximum(acc + b_ref[...], 0.0)


def _tc_matmul(feat, W, bias):
    M = feat.shape[0]
    BM = 512
    return pl.pallas_call(
        _mm_body,
        grid=(M // BM,),
        in_specs=[
            pl.BlockSpec((BM, _SEQ * _C), lambda i: (i, 0)),
            pl.BlockSpec((_SEQ * _C, _C), lambda i: (0, 0)),
            pl.BlockSpec((1, _C), lambda i: (0, 0)),
        ],
        out_specs=pl.BlockSpec((BM, _C), lambda i: (i, 0)),
        out_shape=jax.ShapeDtypeStruct((M, _C), jnp.float32),
    )(feat, W, bias.reshape(1, _C))


def kernel(x, trans_row, trans_col, trans_val, spiral_idx, W, b):
    order = jnp.argsort(trans_row)
    rows = trans_row[order].astype(jnp.int32)
    cols = trans_col[order].astype(jnp.int32)
    vals = trans_val[order].astype(jnp.float32)
    pad = _NNZ_PAD - _NNZ
    rows_p = jnp.concatenate([rows, jnp.zeros((pad,), jnp.int32)])
    cols_p = jnp.concatenate([cols, jnp.zeros((pad,), jnp.int32)])
    vals_p = jnp.concatenate([vals, jnp.zeros((pad,), jnp.float32)])
    edges = jnp.searchsorted(
        rows,
        jnp.minimum(jnp.arange(0, _N_OUT_PAD + _RPW, _RPW), _N_OUT)
        .astype(jnp.int32),
    ).astype(jnp.int32)
    bnd = (jnp.zeros((_NW, 16), jnp.int32)
           .at[:, 0].set(edges[:-1])
           .at[:, 1].set(edges[1:])
           .reshape(-1))

    xf = x.reshape(_B * _N_IN, _C)
    zsrc = jnp.zeros((_RPW, _C), jnp.float32)
    pooled = _sc_pool(xf, cols_p, rows_p, vals_p, bnd, zsrc)

    sidx = spiral_idx.reshape(-1).astype(jnp.int32)
    feat = _sc_gather(pooled, sidx)

    out = _tc_matmul(feat.reshape(_B * _N_OUT, _SEQ * _C), W, b)
    return out.reshape(_B, _N_OUT, _C)
